# baseline (speedup 1.0000x reference)
import functools

import jax
import jax.numpy as jnp
from jax import lax
from jax.experimental import pallas as pl
from jax.experimental.pallas import tpu as pltpu
from jax.experimental.pallas import tpu_sc as plsc

BATCH = 4
NUM_VIRTUAL_TOKENS = 64
NUM_TOKENS = BATCH * NUM_VIRTUAL_TOKENS
ROW_DIM = 98304
NC, NS = 2, 16
NW = NC * NS
CPW = ROW_DIM // NW
H = CPW // 2


def _make_kernel():
    mesh = plsc.VectorSubcoreMesh(core_axis_name="c", subcore_axis_name="s")

    @functools.partial(
        pl.kernel,
        mesh=mesh,
        out_type=jax.ShapeDtypeStruct((NUM_TOKENS, ROW_DIM), jnp.float32),
        compiler_params=pltpu.CompilerParams(needs_layout_passes=False),
        scratch_types=[
            pltpu.VMEM((BATCH, NUM_VIRTUAL_TOKENS), jnp.int32),
            pltpu.SMEM((NUM_TOKENS,), jnp.int32),
            pltpu.VMEM((NUM_VIRTUAL_TOKENS, H), jnp.float32),
            pltpu.SemaphoreType.DMA,
            pltpu.SemaphoreType.DMA,
        ],
    )
    def gather_kernel(prefix_hbm, table_hbm, out_hbm, pvals, psmem, tblk,
                      gsem, wsem):
        wid = lax.axis_index("s") * NC + lax.axis_index("c")
        col0 = wid * CPW

        pltpu.async_copy(table_hbm.at[:, pl.ds(col0, H)], tblk, gsem)

        pltpu.sync_copy(prefix_hbm, pvals)
        lane = lax.iota(jnp.int32, 16)

        for r in range(BATCH):
            for c in range(NUM_VIRTUAL_TOKENS // 16):
                wvec = pvals[r, pl.ds(c * 16, 16)]
                for l in range(16):
                    psmem[(r * NUM_VIRTUAL_TOKENS // 16 + c) * 16 + l] = (
                        jnp.sum(jnp.where(lane == l, wvec, 0)))

        def wait_read():
            pltpu.make_async_copy(
                table_hbm.at[:, pl.ds(0, H)], tblk, gsem).wait()

        def drain_writes():
            for _ in range(4):
                pltpu.make_async_copy(
                    tblk, out_hbm.at[pl.ds(0, 64), pl.ds(0, H)], wsem).wait()

        for h in range(2):
            cbase = col0 + h * H
            wait_read()

            def per_window(win, carry):
                for l in range(16):
                    tok = win * 16 + l
                    pltpu.async_copy(
                        tblk.at[pl.ds(psmem[tok], 1)],
                        out_hbm.at[pl.ds(tok, 1), pl.ds(cbase, H)],
                        wsem,
                    )
                return carry

            lax.fori_loop(0, NUM_TOKENS // 16, per_window, 0)
            drain_writes()
            if h == 0:
                pltpu.async_copy(
                    table_hbm.at[:, pl.ds(col0 + H, H)], tblk, gsem)

    return gather_kernel


_gather = _make_kernel()


def kernel(prefix, embedding):
    out = _gather(prefix.astype(jnp.int32), embedding)
    return out.reshape(BATCH, NUM_VIRTUAL_TOKENS, ROW_DIM)


# speedup vs baseline: 1.0113x; 1.0113x over previous
import functools

import jax
import jax.numpy as jnp
from jax import lax
from jax.experimental import pallas as pl
from jax.experimental.pallas import tpu as pltpu
from jax.experimental.pallas import tpu_sc as plsc

BATCH = 4
NUM_VIRTUAL_TOKENS = 64
NUM_TOKENS = BATCH * NUM_VIRTUAL_TOKENS
ROW_DIM = 98304
NC, NS = 2, 16
NW = NC * NS
CPW = ROW_DIM // NW
H = CPW // 2


def _make_kernel():
    mesh = plsc.VectorSubcoreMesh(core_axis_name="c", subcore_axis_name="s")

    @functools.partial(
        pl.kernel,
        mesh=mesh,
        out_type=jax.ShapeDtypeStruct((NUM_TOKENS, ROW_DIM), jnp.float32),
        compiler_params=pltpu.CompilerParams(needs_layout_passes=False),
        scratch_types=[
            pltpu.VMEM((NUM_TOKENS,), jnp.int32),
            pltpu.SMEM((NUM_TOKENS,), jnp.int32),
            pltpu.VMEM((NUM_VIRTUAL_TOKENS, H), jnp.float32),
            pltpu.SemaphoreType.DMA,
            pltpu.SemaphoreType.DMA,
        ],
    )
    def gather_kernel(prefix_hbm, table_hbm, out_hbm, pvals, psmem, tblk,
                      gsem, wsem):
        wid = lax.axis_index("s") * NC + lax.axis_index("c")
        col0 = wid * CPW

        pltpu.async_copy(table_hbm.at[:, pl.ds(col0, H)], tblk, gsem)

        pltpu.sync_copy(prefix_hbm, pvals)
        lane = lax.iota(jnp.int32, 16)

        def fill_psmem(win, carry):
            wvec = pvals[pl.ds(win * 16, 16)]
            for l in range(16):
                psmem[win * 16 + l] = jnp.sum(jnp.where(lane == l, wvec, 0))
            return carry

        lax.fori_loop(0, NUM_TOKENS // 16, fill_psmem, 0)

        def wait_read():
            pltpu.make_async_copy(
                table_hbm.at[:, pl.ds(0, H)], tblk, gsem).wait()

        def drain_writes():
            for _ in range(4):
                pltpu.make_async_copy(
                    tblk, out_hbm.at[pl.ds(0, 64), pl.ds(0, H)], wsem).wait()

        for h in range(2):
            cbase = col0 + h * H
            wait_read()

            def per_window(win, carry):
                for l in range(16):
                    tok = win * 16 + l
                    pltpu.async_copy(
                        tblk.at[pl.ds(psmem[tok], 1)],
                        out_hbm.at[pl.ds(tok, 1), pl.ds(cbase, H)],
                        wsem,
                    )
                return carry

            lax.fori_loop(0, NUM_TOKENS // 16, per_window, 0)
            drain_writes()
            if h == 0:
                pltpu.async_copy(
                    table_hbm.at[:, pl.ds(col0 + H, H)], tblk, gsem)

    return gather_kernel


_gather = _make_kernel()


def kernel(prefix, embedding):
    p = prefix.reshape(-1).astype(jnp.int32)
    out = _gather(p, embedding)
    return out.reshape(BATCH, NUM_VIRTUAL_TOKENS, ROW_DIM)
